# baseline (speedup 1.0000x reference)
import functools

import jax
import jax.numpy as jnp
from jax import lax
from jax.experimental import pallas as pl
from jax.experimental.pallas import tpu as pltpu
from jax.experimental.pallas import tpu_sc as plsc

_EMB = 128
_SEEDS = 50
_NW = 32
_CHUNK_B = 8


def _ranks_body(randt_ref, rankst_ref):
    rt = randt_ref[...]
    lt = rt[None, :, :] < rt[:, None, :]
    rankst_ref[...] = jnp.sum(lt.astype(jnp.int32), axis=1)


def _row_body(row_ref):
    row_ref[...] = jnp.zeros_like(row_ref)


def _sc_col_body(rankst_hbm, zeros_hbm, col_hbm, buf, rk):
    batch_size = rankst_hbm.shape[1]
    bpw = batch_size // _NW
    nch = bpw // _CHUNK_B
    row_w = _SEEDS * _EMB
    wid = lax.axis_index("s") * 2 + lax.axis_index("c")
    pltpu.sync_copy(zeros_hbm, buf)
    pltpu.sync_copy(rankst_hbm.at[:, pl.ds(wid * bpw, bpw)], rk)
    lane = lax.iota(jnp.int32, 16)
    ones = jnp.full((16,), 1.0, jnp.float32)
    zero = jnp.zeros((16,), jnp.float32)
    for ch in range(nch):
        base_b = wid * bpw + ch * _CHUNK_B
        for val in (ones, zero):
            for lb in range(_CHUNK_B):
                lb32 = jnp.full((16,), ch * _CHUNK_B + lb, jnp.int32)
                for cc in range(4):
                    c = cc * 16 + lane
                    c_clamped = jnp.minimum(c, _SEEDS - 1)
                    rvec = plsc.load_gather(rk, [c_clamped, lb32])
                    offs = (lb * _SEEDS + rvec) * _EMB + c
                    plsc.store_scatter(buf, [offs], val, mask=c < _SEEDS)
            if val is ones:
                pltpu.sync_copy(
                    buf, col_hbm.at[pl.ds(base_b * row_w, _CHUNK_B * row_w)]
                )


def kernel(problems):
    batch_size, job_cnt, machine_cnt = problems.shape
    seed_cnt = max(machine_cnt, _SEEDS)
    rand = jax.random.uniform(
        jax.random.key(42), (batch_size, seed_cnt), dtype=jnp.float32
    )
    rand_t = rand.T
    ranks_t = pl.pallas_call(
        _ranks_body,
        out_shape=jax.ShapeDtypeStruct((seed_cnt, batch_size), jnp.int32),
    )(rand_t)
    sc_col = functools.partial(
        pl.kernel,
        mesh=plsc.VectorSubcoreMesh(core_axis_name="c", subcore_axis_name="s"),
        out_type=jax.ShapeDtypeStruct(
            (batch_size * machine_cnt * _EMB,), jnp.float32
        ),
        scratch_types=[
            pltpu.VMEM((_CHUNK_B * machine_cnt * _EMB,), jnp.float32),
            pltpu.VMEM((seed_cnt, batch_size // _NW), jnp.int32),
        ],
        compiler_params=pltpu.CompilerParams(
            needs_layout_passes=False, use_tc_tiling_on_sc=False
        ),
    )(_sc_col_body)
    zchunk = jnp.zeros((_CHUNK_B * machine_cnt * _EMB,), jnp.float32)
    col_emb = sc_col(ranks_t, zchunk).reshape(
        batch_size, machine_cnt, _EMB
    )

    blk = 32
    row_emb = pl.pallas_call(
        _row_body,
        grid=(batch_size // blk,),
        out_specs=pl.BlockSpec((blk, job_cnt, _EMB), lambda i: (i, 0, 0)),
        out_shape=jax.ShapeDtypeStruct(
            (batch_size, job_cnt, _EMB), jnp.float32
        ),
    )()
    return (row_emb, col_emb)


# speedup vs baseline: 1.3827x; 1.3827x over previous
import functools

import jax
import jax.numpy as jnp
from jax import lax
from jax.experimental import pallas as pl
from jax.experimental.pallas import tpu as pltpu
from jax.experimental.pallas import tpu_sc as plsc

_EMB = 128
_SEEDS = 50
_NW = 32
_ZBUF = 81920


def _ranks_body(randt_ref, rankst_ref):
    rt = randt_ref[...]
    lt = rt[None, :, :] < rt[:, None, :]
    rankst_ref[...] = jnp.sum(lt.astype(jnp.int32), axis=1)


def _col_body(ranks_ref, col_ref):
    ranks = ranks_ref[...]
    m = jax.lax.broadcasted_iota(jnp.int32, (ranks.shape[0], _SEEDS, _EMB), 1)
    col_ref[...] = (ranks[:, None, :] == m).astype(jnp.float32)


def _sc_row_body(row_hbm, zbuf):
    per_w = row_hbm.shape[0] // _NW
    nch = per_w // _ZBUF
    wid = lax.axis_index("s") * 2 + lax.axis_index("c")
    zero = jnp.zeros((16,), jnp.float32)

    def _z(i, _):
        zbuf[pl.ds(i * 16, 16)] = zero
        return _

    lax.fori_loop(0, _ZBUF // 16, _z, 0)
    base = wid * per_w
    for i in range(nch):
        pltpu.sync_copy(zbuf, row_hbm.at[pl.ds(base + i * _ZBUF, _ZBUF)])


def kernel(problems):
    batch_size, job_cnt, machine_cnt = problems.shape
    seed_cnt = max(machine_cnt, _SEEDS)

    sc_row = functools.partial(
        pl.kernel,
        mesh=plsc.VectorSubcoreMesh(core_axis_name="c", subcore_axis_name="s"),
        out_type=jax.ShapeDtypeStruct(
            (batch_size * job_cnt * _EMB,), jnp.float32
        ),
        scratch_types=[pltpu.VMEM((_ZBUF,), jnp.float32)],
        compiler_params=pltpu.CompilerParams(needs_layout_passes=False),
    )(_sc_row_body)
    row_emb = sc_row().reshape(batch_size, job_cnt, _EMB)

    rand = jax.random.uniform(
        jax.random.key(42), (batch_size, seed_cnt), dtype=jnp.float32
    )
    rand_t = rand.T
    ranks_t = pl.pallas_call(
        _ranks_body,
        out_shape=jax.ShapeDtypeStruct((seed_cnt, batch_size), jnp.int32),
    )(rand_t)
    ranks = jnp.pad(
        ranks_t.T, ((0, 0), (0, _EMB - seed_cnt)), constant_values=127
    )
    blk = 64
    col_emb = pl.pallas_call(
        _col_body,
        grid=(batch_size // blk,),
        in_specs=[pl.BlockSpec((blk, _EMB), lambda i: (i, 0))],
        out_specs=pl.BlockSpec((blk, machine_cnt, _EMB), lambda i: (i, 0, 0)),
        out_shape=jax.ShapeDtypeStruct(
            (batch_size, machine_cnt, _EMB), jnp.float32
        ),
    )(ranks)
    return (row_emb, col_emb)
